# Initial kernel scaffold; baseline (speedup 1.0000x reference)
import jax
import jax.numpy as jnp
from jax.experimental import pallas as pl


def kernel(x, edge_index, edge_attr, batch, smiles, Wl1, bl1, Wr1, br1, We1, att1, cb1, Wl2, bl2, Wr2, br2, We2, att2, cb2, L1W, L1b, L2W, L2b):
    raise NotImplementedError("write your pallas kernel here")


# speedup vs baseline: 2.5948x; 2.5948x over previous
import functools
import jax
import jax.numpy as jnp
from jax import lax
from jax.experimental import pallas as pl
from jax.experimental.pallas import tpu as pltpu
from jax.experimental.pallas import tpu_sc as plsc

N = 10000
E = 160000
G = 64
D = 256

NC = 2
NS = 16
NW = NC * NS

NB = 25
BN = N // NB
EB = 25
BE = E // EB
A = E + N
AB = 25
BA = A // AB

CH = 200
CHS = 40
NPAD = 10240

_mesh = plsc.VectorSubcoreMesh(core_axis_name="c", subcore_axis_name="s")


def _sc_gather(table, idx, d):
    b_per_w = E // NW
    steps = b_per_w // CH

    @functools.partial(
        pl.kernel, mesh=_mesh,
        out_type=jax.ShapeDtypeStruct((E, d), jnp.float32),
        scratch_types=[
            pltpu.VMEM((CH,), jnp.int32),
            pltpu.VMEM((CH, d), jnp.float32),
            pltpu.SemaphoreType.DMA,
        ],
    )
    def k(table_hbm, idx_hbm, out_hbm, idx_v, rows_v, sem):
        wid = lax.axis_index("s") * NC + lax.axis_index("c")

        def body(j, _):
            base = wid * b_per_w + j * CH
            pltpu.sync_copy(idx_hbm.at[pl.ds(base, CH)], idx_v)
            pltpu.async_copy(table_hbm.at[idx_v], rows_v, sem).wait()
            pltpu.sync_copy(rows_v, out_hbm.at[pl.ds(base, CH)])
            return 0

        lax.fori_loop(0, steps, body, 0)

    return k(table, idx)


def _sc_scatter_add(vals, idx, d):
    assert d == 128
    e_per_core = E // NC
    e_per_sub = e_per_core // NS
    steps = e_per_sub // CHS
    rows_per_sub = NPAD // NS

    @functools.partial(
        pl.kernel, mesh=_mesh,
        out_type=jax.ShapeDtypeStruct((NC, NPAD, d), jnp.float32),
        scratch_types=[
            pltpu.VMEM((CHS,), jnp.int32),
            pltpu.VMEM((CHS, d), jnp.float32),
            pltpu.VMEM_SHARED((NPAD, d), jnp.float32),
        ],
    )
    def k(vals_hbm, idx_hbm, zeros_hbm, out_hbm, idx_v, vals_v, acc_sh):
        cid = lax.axis_index("c")
        sid = lax.axis_index("s")

        pltpu.sync_copy(zeros_hbm.at[pl.ds(sid * rows_per_sub, rows_per_sub)],
                        acc_sh.at[pl.ds(sid * rows_per_sub, rows_per_sub)])
        plsc.subcore_barrier()

        def body(j, _):
            base = cid * e_per_core + sid * e_per_sub + j * CHS
            pltpu.sync_copy(idx_hbm.at[pl.ds(base, CHS)], idx_v)
            pltpu.sync_copy(vals_hbm.at[pl.ds(base, CHS)], vals_v)
            pltpu.sync_copy(vals_v, acc_sh.at[idx_v], add=True)
            return 0

        lax.fori_loop(0, steps, body, 0)
        plsc.subcore_barrier()

        pltpu.sync_copy(acc_sh.at[pl.ds(sid * rows_per_sub, rows_per_sub)],
                        out_hbm.at[cid, pl.ds(sid * rows_per_sub, rows_per_sub)])

    zeros = jnp.zeros((NPAD, d), jnp.float32)
    return k(vals, idx, zeros)


def _mm_body(x_ref, w_ref, b_ref, o_ref):
    o_ref[...] = lax.dot_general(
        x_ref[...], w_ref[...], (((1,), (0,)), ((), ())),
        preferred_element_type=jnp.float32) + b_ref[...]


def _mm(x, w, b, rows, block):
    grid = rows // block
    return pl.pallas_call(
        _mm_body,
        grid=(grid,),
        in_specs=[
            pl.BlockSpec((block, w.shape[0]), lambda i: (i, 0)),
            pl.BlockSpec(w.shape, lambda i: (0, 0)),
            pl.BlockSpec((1, w.shape[1]), lambda i: (0, 0)),
        ],
        out_specs=pl.BlockSpec((block, w.shape[1]), lambda i: (i, 0)),
        out_shape=jax.ShapeDtypeStruct((rows, w.shape[1]), jnp.float32),
    )(x, w, b)


def _edge_alpha_body(rl_ref, rr_ref, ea_ref, we_ref, att_ref, o_ref):
    ec = lax.dot_general(ea_ref[...], we_ref[...], (((1,), (0,)), ((), ())),
                         preferred_element_type=jnp.float32)
    m = rl_ref[...] + rr_ref[...] + ec
    m = jnp.where(m > 0, m, 0.2 * m)
    o_ref[...] = lax.dot_general(m, att_ref[...], (((1,), (0,)), ((), ())),
                                 preferred_element_type=jnp.float32)


def _edge_alpha(rl, rr, eap, wep, attc):
    return pl.pallas_call(
        _edge_alpha_body,
        grid=(EB,),
        in_specs=[
            pl.BlockSpec((BE, D), lambda i: (i, 0)),
            pl.BlockSpec((BE, D), lambda i: (i, 0)),
            pl.BlockSpec((BE, 128), lambda i: (i, 0)),
            pl.BlockSpec((128, D), lambda i: (0, 0)),
            pl.BlockSpec((D, 1), lambda i: (0, 0)),
        ],
        out_specs=pl.BlockSpec((BE, 1), lambda i: (i, 0)),
        out_shape=jax.ShapeDtypeStruct((E, 1), jnp.float32),
    )(rl, rr, eap, wep, attc)


def _self_alpha_body(xl_ref, xr_ref, la_ref, we_ref, att_ref, o_ref):
    la = la_ref[0] + la_ref[1]
    deg = jnp.maximum(la[:, 11:12], 1.0)
    lam = la / deg
    colmask = (lax.broadcasted_iota(jnp.int32, (1, 128), 1) < 11).astype(jnp.float32)
    lam = lam * colmask
    ec = lax.dot_general(lam, we_ref[...], (((1,), (0,)), ((), ())),
                         preferred_element_type=jnp.float32)
    m = xl_ref[...] + xr_ref[...] + ec
    m = jnp.where(m > 0, m, 0.2 * m)
    o_ref[...] = lax.dot_general(m, att_ref[...], (((1,), (0,)), ((), ())),
                                 preferred_element_type=jnp.float32)


def _self_alpha(xl, xr, la, wep, attc):
    return pl.pallas_call(
        _self_alpha_body,
        grid=(NB,),
        in_specs=[
            pl.BlockSpec((BN, D), lambda i: (i, 0)),
            pl.BlockSpec((BN, D), lambda i: (i, 0)),
            pl.BlockSpec((NC, BN, 128), lambda i: (0, i, 0)),
            pl.BlockSpec((128, D), lambda i: (0, 0)),
            pl.BlockSpec((D, 1), lambda i: (0, 0)),
        ],
        out_specs=pl.BlockSpec((BN, 1), lambda i: (i, 0)),
        out_shape=jax.ShapeDtypeStruct((N, 1), jnp.float32),
    )(xl, xr, la, wep, attc)


def _amax_body(a_ref, o_ref):
    @pl.when(pl.program_id(0) == 0)
    def _():
        o_ref[...] = jnp.full((1, 1), -1e30, jnp.float32)
    o_ref[...] = jnp.maximum(o_ref[...], jnp.max(a_ref[...])[None, None])


def _amax(a):
    return pl.pallas_call(
        _amax_body,
        grid=(AB,),
        in_specs=[pl.BlockSpec((BA, 1), lambda i: (i, 0))],
        out_specs=pl.BlockSpec((1, 1), lambda i: (0, 0)),
        out_shape=jax.ShapeDtypeStruct((1, 1), jnp.float32),
    )(a)


def _exp_body(a_ref, mx_ref, o_ref):
    o_ref[...] = jnp.exp(a_ref[...] - mx_ref[0, 0])


def _exp(a, mx):
    return pl.pallas_call(
        _exp_body,
        grid=(AB,),
        in_specs=[
            pl.BlockSpec((BA, 1), lambda i: (i, 0)),
            pl.BlockSpec((1, 1), lambda i: (0, 0)),
        ],
        out_specs=pl.BlockSpec((BA, 1), lambda i: (i, 0)),
        out_shape=jax.ShapeDtypeStruct((A, 1), jnp.float32),
    )(a, mx)


def _apply_body(rl_ref, ex_ref, v_ref, exb_ref):
    ex = ex_ref[...]
    v_ref[...] = rl_ref[...] * ex
    col0 = lax.broadcasted_iota(jnp.int32, (BE, 128), 1) == 0
    exb_ref[...] = jnp.where(col0, jnp.broadcast_to(ex, (BE, 128)), 0.0)


def _apply(rl, ex_e):
    return pl.pallas_call(
        _apply_body,
        grid=(EB,),
        in_specs=[
            pl.BlockSpec((BE, D), lambda i: (i, 0)),
            pl.BlockSpec((BE, 1), lambda i: (i, 0)),
        ],
        out_specs=[
            pl.BlockSpec((BE, D), lambda i: (i, 0)),
            pl.BlockSpec((BE, 128), lambda i: (i, 0)),
        ],
        out_shape=[
            jax.ShapeDtypeStruct((E, D), jnp.float32),
            jax.ShapeDtypeStruct((E, 128), jnp.float32),
        ],
    )(rl, ex_e)


def _final_body(x_ref, xl_ref, s0_ref, s1_ref, dn_ref, exs_ref, cb_ref, o_ref):
    seg = jnp.concatenate([s0_ref[0] + s0_ref[1], s1_ref[0] + s1_ref[1]], axis=1)
    exs = exs_ref[...]
    num = seg + exs * xl_ref[...]
    denom = dn_ref[0, :, 0:1] + dn_ref[1, :, 0:1] + exs + 1e-16
    out = num / denom + cb_ref[...]
    h = x_ref[...] + out
    o_ref[...] = jnp.where(h > 0, h, 0.01 * h)


def _final(x, xl, s0, s1, dn, exs, cb2d):
    return pl.pallas_call(
        _final_body,
        grid=(NB,),
        in_specs=[
            pl.BlockSpec((BN, D), lambda i: (i, 0)),
            pl.BlockSpec((BN, D), lambda i: (i, 0)),
            pl.BlockSpec((NC, BN, D // 2), lambda i: (0, i, 0)),
            pl.BlockSpec((NC, BN, D // 2), lambda i: (0, i, 0)),
            pl.BlockSpec((NC, BN, 128), lambda i: (0, i, 0)),
            pl.BlockSpec((BN, 1), lambda i: (i, 0)),
            pl.BlockSpec((1, D), lambda i: (0, 0)),
        ],
        out_specs=pl.BlockSpec((BN, D), lambda i: (i, 0)),
        out_shape=jax.ShapeDtypeStruct((N, D), jnp.float32),
    )(x, xl, s0, s1, dn, exs, cb2d)


def _pool_body(h_ref, b_ref, sum_ref, cnt_ref, max_ref):
    @pl.when(pl.program_id(0) == 0)
    def _():
        sum_ref[...] = jnp.zeros_like(sum_ref)
        cnt_ref[...] = jnp.zeros_like(cnt_ref)
        max_ref[...] = jnp.full_like(max_ref, -1e30)

    b = b_ref[...]
    onehot = (b == lax.broadcasted_iota(jnp.int32, (BN, G), 1)).astype(jnp.float32)
    h = h_ref[...]
    sum_ref[...] += lax.dot_general(onehot, h, (((0,), (0,)), ((), ())),
                                    preferred_element_type=jnp.float32)
    ones = jnp.ones((BN, 1), jnp.float32)
    cnt_ref[...] += lax.dot_general(onehot, ones, (((0,), (0,)), ((), ())),
                                    preferred_element_type=jnp.float32)

    def body(g, _):
        mask = b == g
        mx = jnp.max(jnp.where(mask, h, -1e30), axis=0, keepdims=True)
        max_ref[pl.ds(g, 1), :] = jnp.maximum(max_ref[pl.ds(g, 1), :], mx)
        return 0

    lax.fori_loop(0, G, body, 0)


def _pool(h, b2d):
    return pl.pallas_call(
        _pool_body,
        grid=(NB,),
        in_specs=[
            pl.BlockSpec((BN, D), lambda i: (i, 0)),
            pl.BlockSpec((BN, 1), lambda i: (i, 0)),
        ],
        out_specs=[
            pl.BlockSpec((G, D), lambda i: (0, 0)),
            pl.BlockSpec((G, 1), lambda i: (0, 0)),
            pl.BlockSpec((G, D), lambda i: (0, 0)),
        ],
        out_shape=[
            jax.ShapeDtypeStruct((G, D), jnp.float32),
            jax.ShapeDtypeStruct((G, 1), jnp.float32),
            jax.ShapeDtypeStruct((G, D), jnp.float32),
        ],
    )(h, b2d)


def _head_body(s1_ref, c1_ref, m1_ref, s2_ref, c2_ref, m2_ref,
               w1_ref, b1_ref, w2_ref, b2_ref, o_ref):
    def pooled(s_ref, c_ref, m_ref):
        cnt = c_ref[...]
        mean = s_ref[...] / jnp.maximum(cnt, 1.0)
        mx = jnp.where(cnt > 0, m_ref[...], 0.0)
        return jnp.concatenate([mean, mx], axis=1)

    z = pooled(s1_ref, c1_ref, m1_ref) + pooled(s2_ref, c2_ref, m2_ref)
    z = lax.dot_general(z, w1_ref[...], (((1,), (0,)), ((), ())),
                        preferred_element_type=jnp.float32) + b1_ref[...]
    z = jnp.where(z > 0, z, 0.01 * z)
    o_ref[...] = lax.dot_general(z, w2_ref[...], (((1,), (0,)), ((), ())),
                                 preferred_element_type=jnp.float32) + b2_ref[...]


def _head(s1, c1, m1, s2, c2, m2, w1, b1, w2, b2):
    full = lambda a: pl.BlockSpec(a.shape, lambda: (0,) * a.ndim)
    return pl.pallas_call(
        _head_body,
        in_specs=[full(a) for a in (s1, c1, m1, s2, c2, m2, w1, b1, w2, b2)],
        out_specs=pl.BlockSpec((G, 170), lambda: (0, 0)),
        out_shape=jax.ShapeDtypeStruct((G, 170), jnp.float32),
    )(s1, c1, m1, s2, c2, m2, w1, b1, w2, b2)


def _gat_layer(x, src, dst, eap, la_vals, Wl, bl, Wr, br, WeP, attc, cb2d):
    xl = _mm(x, Wl, bl, N, BN)
    xr = _mm(x, Wr, br, N, BN)

    la = _sc_scatter_add(la_vals, dst, 128)
    rl = _sc_gather(xl, src, D)
    rr = _sc_gather(xr, dst, D)

    alpha_e = _edge_alpha(rl, rr, eap, WeP, attc)
    alpha_s = _self_alpha(xl, xr, la, WeP, attc)

    alpha = jnp.concatenate([alpha_e, alpha_s], axis=0)
    mx = _amax(alpha)
    ex = _exp(alpha, mx)
    ex_e, ex_s = ex[:E], ex[E:]

    v, exb = _apply(rl, ex_e)
    dn = _sc_scatter_add(exb, dst, 128)
    s0 = _sc_scatter_add(v[:, :D // 2], dst, D // 2)
    s1 = _sc_scatter_add(v[:, D // 2:], dst, D // 2)

    return _final(x, xl, s0, s1, dn, ex_s, cb2d)


def kernel(x, edge_index, edge_attr, batch, smiles, Wl1, bl1, Wr1, br1, We1,
           att1, cb1, Wl2, bl2, Wr2, br2, We2, att2, cb2, L1W, L1b, L2W, L2b):
    src = edge_index[0].astype(jnp.int32)
    dst = edge_index[1].astype(jnp.int32)
    ea = edge_attr.astype(jnp.float32)
    la_vals = jnp.concatenate(
        [ea, jnp.ones((E, 1), jnp.float32), jnp.zeros((E, 116), jnp.float32)], axis=1)
    eap = la_vals
    b2d = batch.astype(jnp.int32).reshape(N, 1)

    def prep(We, att, b_l, b_r, cb):
        return (jnp.pad(We, ((0, 117), (0, 0))), att.reshape(D, 1),
                b_l.reshape(1, D), b_r.reshape(1, D), cb.reshape(1, D))

    WeP1, attc1, bl1r, br1r, cb1r = prep(We1, att1, bl1, br1, cb1)
    WeP2, attc2, bl2r, br2r, cb2r = prep(We2, att2, bl2, br2, cb2)

    h1 = _gat_layer(x, src, dst, eap, la_vals, Wl1, bl1r, Wr1, br1r, WeP1, attc1, cb1r)
    s1, c1, m1 = _pool(h1, b2d)
    h2 = _gat_layer(h1, src, dst, eap, la_vals, Wl2, bl2r, Wr2, br2r, WeP2, attc2, cb2r)
    s2, c2, m2 = _pool(h2, b2d)

    return _head(s1, c1, m1, s2, c2, m2, L1W, L1b.reshape(1, D), L2W, L2b.reshape(1, 170))


# speedup vs baseline: 2.5950x; 1.0001x over previous
import functools
import jax
import jax.numpy as jnp
from jax import lax
from jax.experimental import pallas as pl
from jax.experimental.pallas import tpu as pltpu
from jax.experimental.pallas import tpu_sc as plsc

N = 10000
E = 160000
G = 64
D = 256

NC = 2
NS = 16
NW = NC * NS

NB = 25
BN = N // NB
EB = 25
BE = E // EB
A = E + N
AB = 25
BA = A // AB

CH = 200
CHS = 40
NPAD = 10240

_mesh = plsc.VectorSubcoreMesh(core_axis_name="c", subcore_axis_name="s")


def _sc_gather(table, idx, d):
    b_per_w = E // NW
    steps = b_per_w // CH

    @functools.partial(
        pl.kernel, mesh=_mesh,
        out_type=jax.ShapeDtypeStruct((E, d), jnp.float32),
        scratch_types=[
            pltpu.VMEM((CH,), jnp.int32),
            pltpu.VMEM((CH, d), jnp.float32),
            pltpu.SemaphoreType.DMA,
        ],
    )
    def k(table_hbm, idx_hbm, out_hbm, idx_v, rows_v, sem):
        wid = lax.axis_index("s") * NC + lax.axis_index("c")

        def body(j, _):
            base = wid * b_per_w + j * CH
            pltpu.sync_copy(idx_hbm.at[pl.ds(base, CH)], idx_v)
            pltpu.async_copy(table_hbm.at[idx_v], rows_v, sem).wait()
            pltpu.sync_copy(rows_v, out_hbm.at[pl.ds(base, CH)])
            return 0

        lax.fori_loop(0, steps, body, 0)

    return k(table, idx)


def _sc_scatter_add(vals, idx, d):
    assert d == 128
    e_per_core = E // NC
    e_per_sub = e_per_core // NS
    steps = e_per_sub // CHS
    rows_per_sub = NPAD // NS

    @functools.partial(
        pl.kernel, mesh=_mesh,
        out_type=jax.ShapeDtypeStruct((NC, NPAD, d), jnp.float32),
        scratch_types=[
            pltpu.VMEM((CHS,), jnp.int32),
            pltpu.VMEM((CHS, d), jnp.float32),
            pltpu.VMEM_SHARED((NPAD, d), jnp.float32),
        ],
    )
    def k(vals_hbm, idx_hbm, zeros_hbm, out_hbm, idx_v, vals_v, acc_sh):
        cid = lax.axis_index("c")
        sid = lax.axis_index("s")

        pltpu.sync_copy(zeros_hbm.at[pl.ds(sid * rows_per_sub, rows_per_sub)],
                        acc_sh.at[pl.ds(sid * rows_per_sub, rows_per_sub)])
        plsc.subcore_barrier()

        def body(j, _):
            base = cid * e_per_core + sid * e_per_sub + j * CHS
            pltpu.sync_copy(idx_hbm.at[pl.ds(base, CHS)], idx_v)
            pltpu.sync_copy(vals_hbm.at[pl.ds(base, CHS)], vals_v)
            pltpu.sync_copy(vals_v, acc_sh.at[idx_v], add=True)
            return 0

        lax.fori_loop(0, steps, body, 0)
        plsc.subcore_barrier()

        pltpu.sync_copy(acc_sh.at[pl.ds(sid * rows_per_sub, rows_per_sub)],
                        out_hbm.at[cid, pl.ds(sid * rows_per_sub, rows_per_sub)])

    zeros = jnp.zeros((NPAD, d), jnp.float32)
    return k(vals, idx, zeros)


def _mm_body(x_ref, w_ref, b_ref, o_ref):
    o_ref[...] = lax.dot_general(
        x_ref[...], w_ref[...], (((1,), (0,)), ((), ())),
        preferred_element_type=jnp.float32) + b_ref[...]


def _mm(x, w, b, rows, block):
    grid = rows // block
    return pl.pallas_call(
        _mm_body,
        grid=(grid,),
        in_specs=[
            pl.BlockSpec((block, w.shape[0]), lambda i: (i, 0)),
            pl.BlockSpec(w.shape, lambda i: (0, 0)),
            pl.BlockSpec((1, w.shape[1]), lambda i: (0, 0)),
        ],
        out_specs=pl.BlockSpec((block, w.shape[1]), lambda i: (i, 0)),
        out_shape=jax.ShapeDtypeStruct((rows, w.shape[1]), jnp.float32),
    )(x, w, b)


def _edge_alpha_body(rl_ref, rr_ref, ea_ref, we_ref, att_ref, o_ref):
    ec = lax.dot_general(ea_ref[...], we_ref[...], (((1,), (0,)), ((), ())),
                         preferred_element_type=jnp.float32)
    m = rl_ref[...] + rr_ref[...] + ec
    m = jnp.where(m > 0, m, 0.2 * m)
    o_ref[...] = lax.dot_general(m, att_ref[...], (((1,), (0,)), ((), ())),
                                 preferred_element_type=jnp.float32)


def _edge_alpha(rl, rr, eap, wep, attc):
    return pl.pallas_call(
        _edge_alpha_body,
        grid=(EB,),
        in_specs=[
            pl.BlockSpec((BE, D), lambda i: (i, 0)),
            pl.BlockSpec((BE, D), lambda i: (i, 0)),
            pl.BlockSpec((BE, 128), lambda i: (i, 0)),
            pl.BlockSpec((128, D), lambda i: (0, 0)),
            pl.BlockSpec((D, 1), lambda i: (0, 0)),
        ],
        out_specs=pl.BlockSpec((BE, 1), lambda i: (i, 0)),
        out_shape=jax.ShapeDtypeStruct((E, 1), jnp.float32),
    )(rl, rr, eap, wep, attc)


def _self_alpha_body(xl_ref, xr_ref, la_ref, we_ref, att_ref, o_ref):
    la = la_ref[0] + la_ref[1]
    deg = jnp.maximum(la[:, 11:12], 1.0)
    lam = la / deg
    colmask = (lax.broadcasted_iota(jnp.int32, (1, 128), 1) < 11).astype(jnp.float32)
    lam = lam * colmask
    ec = lax.dot_general(lam, we_ref[...], (((1,), (0,)), ((), ())),
                         preferred_element_type=jnp.float32)
    m = xl_ref[...] + xr_ref[...] + ec
    m = jnp.where(m > 0, m, 0.2 * m)
    o_ref[...] = lax.dot_general(m, att_ref[...], (((1,), (0,)), ((), ())),
                                 preferred_element_type=jnp.float32)


def _self_alpha(xl, xr, la, wep, attc):
    return pl.pallas_call(
        _self_alpha_body,
        grid=(NB,),
        in_specs=[
            pl.BlockSpec((BN, D), lambda i: (i, 0)),
            pl.BlockSpec((BN, D), lambda i: (i, 0)),
            pl.BlockSpec((NC, BN, 128), lambda i: (0, i, 0)),
            pl.BlockSpec((128, D), lambda i: (0, 0)),
            pl.BlockSpec((D, 1), lambda i: (0, 0)),
        ],
        out_specs=pl.BlockSpec((BN, 1), lambda i: (i, 0)),
        out_shape=jax.ShapeDtypeStruct((N, 1), jnp.float32),
    )(xl, xr, la, wep, attc)


def _amax_body(a_ref, o_ref):
    @pl.when(pl.program_id(0) == 0)
    def _():
        o_ref[...] = jnp.full((1, 1), -1e30, jnp.float32)
    o_ref[...] = jnp.maximum(o_ref[...], jnp.max(a_ref[...])[None, None])


def _amax(a):
    return pl.pallas_call(
        _amax_body,
        grid=(AB,),
        in_specs=[pl.BlockSpec((BA, 1), lambda i: (i, 0))],
        out_specs=pl.BlockSpec((1, 1), lambda i: (0, 0)),
        out_shape=jax.ShapeDtypeStruct((1, 1), jnp.float32),
    )(a)


def _exp_body(a_ref, mx_ref, o_ref):
    o_ref[...] = jnp.exp(a_ref[...] - mx_ref[0, 0])


def _exp(a, mx):
    return pl.pallas_call(
        _exp_body,
        grid=(AB,),
        in_specs=[
            pl.BlockSpec((BA, 1), lambda i: (i, 0)),
            pl.BlockSpec((1, 1), lambda i: (0, 0)),
        ],
        out_specs=pl.BlockSpec((BA, 1), lambda i: (i, 0)),
        out_shape=jax.ShapeDtypeStruct((A, 1), jnp.float32),
    )(a, mx)


def _apply_body(rl_ref, ex_ref, v_ref, exb_ref):
    ex = ex_ref[...]
    v_ref[...] = rl_ref[...] * ex
    col0 = lax.broadcasted_iota(jnp.int32, (BE, 128), 1) == 0
    exb_ref[...] = jnp.where(col0, jnp.broadcast_to(ex, (BE, 128)), 0.0)


def _apply(rl, ex_e):
    return pl.pallas_call(
        _apply_body,
        grid=(EB,),
        in_specs=[
            pl.BlockSpec((BE, D), lambda i: (i, 0)),
            pl.BlockSpec((BE, 1), lambda i: (i, 0)),
        ],
        out_specs=[
            pl.BlockSpec((BE, D), lambda i: (i, 0)),
            pl.BlockSpec((BE, 128), lambda i: (i, 0)),
        ],
        out_shape=[
            jax.ShapeDtypeStruct((E, D), jnp.float32),
            jax.ShapeDtypeStruct((E, 128), jnp.float32),
        ],
    )(rl, ex_e)


def _final_body(x_ref, xl_ref, s0_ref, s1_ref, dn_ref, exs_ref, cb_ref, o_ref):
    seg = jnp.concatenate([s0_ref[0] + s0_ref[1], s1_ref[0] + s1_ref[1]], axis=1)
    exs = exs_ref[...]
    num = seg + exs * xl_ref[...]
    denom = dn_ref[0, :, 0:1] + dn_ref[1, :, 0:1] + exs + 1e-16
    out = num / denom + cb_ref[...]
    h = x_ref[...] + out
    o_ref[...] = jnp.where(h > 0, h, 0.01 * h)


def _final(x, xl, s0, s1, dn, exs, cb2d):
    return pl.pallas_call(
        _final_body,
        grid=(NB,),
        in_specs=[
            pl.BlockSpec((BN, D), lambda i: (i, 0)),
            pl.BlockSpec((BN, D), lambda i: (i, 0)),
            pl.BlockSpec((NC, BN, D // 2), lambda i: (0, i, 0)),
            pl.BlockSpec((NC, BN, D // 2), lambda i: (0, i, 0)),
            pl.BlockSpec((NC, BN, 128), lambda i: (0, i, 0)),
            pl.BlockSpec((BN, 1), lambda i: (i, 0)),
            pl.BlockSpec((1, D), lambda i: (0, 0)),
        ],
        out_specs=pl.BlockSpec((BN, D), lambda i: (i, 0)),
        out_shape=jax.ShapeDtypeStruct((N, D), jnp.float32),
    )(x, xl, s0, s1, dn, exs, cb2d)


def _pool_body(h_ref, b_ref, sum_ref, cnt_ref, max_ref):
    @pl.when(pl.program_id(0) == 0)
    def _():
        sum_ref[...] = jnp.zeros_like(sum_ref)
        cnt_ref[...] = jnp.zeros_like(cnt_ref)
        max_ref[...] = jnp.full_like(max_ref, -1e30)

    b = b_ref[...]
    onehot = (b == lax.broadcasted_iota(jnp.int32, (BN, G), 1)).astype(jnp.float32)
    h = h_ref[...]
    sum_ref[...] += lax.dot_general(onehot, h, (((0,), (0,)), ((), ())),
                                    preferred_element_type=jnp.float32)
    ones = jnp.ones((BN, 1), jnp.float32)
    cnt_ref[...] += lax.dot_general(onehot, ones, (((0,), (0,)), ((), ())),
                                    preferred_element_type=jnp.float32)

    def body(g, _):
        mask = b == g
        mx = jnp.max(jnp.where(mask, h, -1e30), axis=0, keepdims=True)
        max_ref[pl.ds(g, 1), :] = jnp.maximum(max_ref[pl.ds(g, 1), :], mx)
        return 0

    lax.fori_loop(0, G, body, 0)


def _pool(h, b2d):
    return pl.pallas_call(
        _pool_body,
        grid=(NB,),
        in_specs=[
            pl.BlockSpec((BN, D), lambda i: (i, 0)),
            pl.BlockSpec((BN, 1), lambda i: (i, 0)),
        ],
        out_specs=[
            pl.BlockSpec((G, D), lambda i: (0, 0)),
            pl.BlockSpec((G, 1), lambda i: (0, 0)),
            pl.BlockSpec((G, D), lambda i: (0, 0)),
        ],
        out_shape=[
            jax.ShapeDtypeStruct((G, D), jnp.float32),
            jax.ShapeDtypeStruct((G, 1), jnp.float32),
            jax.ShapeDtypeStruct((G, D), jnp.float32),
        ],
    )(h, b2d)


def _head_body(s1_ref, c1_ref, m1_ref, s2_ref, c2_ref, m2_ref,
               w1_ref, b1_ref, w2_ref, b2_ref, o_ref):
    def pooled(s_ref, c_ref, m_ref):
        cnt = c_ref[...]
        mean = s_ref[...] / jnp.maximum(cnt, 1.0)
        mx = jnp.where(cnt > 0, m_ref[...], 0.0)
        return jnp.concatenate([mean, mx], axis=1)

    z = pooled(s1_ref, c1_ref, m1_ref) + pooled(s2_ref, c2_ref, m2_ref)
    z = lax.dot_general(z, w1_ref[...], (((1,), (0,)), ((), ())),
                        preferred_element_type=jnp.float32) + b1_ref[...]
    z = jnp.where(z > 0, z, 0.01 * z)
    o_ref[...] = lax.dot_general(z, w2_ref[...], (((1,), (0,)), ((), ())),
                                 preferred_element_type=jnp.float32) + b2_ref[...]


def _head(s1, c1, m1, s2, c2, m2, w1, b1, w2, b2):
    full = lambda a: pl.BlockSpec(a.shape, lambda: (0,) * a.ndim)
    return pl.pallas_call(
        _head_body,
        in_specs=[full(a) for a in (s1, c1, m1, s2, c2, m2, w1, b1, w2, b2)],
        out_specs=pl.BlockSpec((G, 170), lambda: (0, 0)),
        out_shape=jax.ShapeDtypeStruct((G, 170), jnp.float32),
    )(s1, c1, m1, s2, c2, m2, w1, b1, w2, b2)


def _gat_layer(x, src, dst, eap, la, Wl, bl, Wr, br, WeP, attc, cb2d):
    xl = _mm(x, Wl, bl, N, BN)
    xr = _mm(x, Wr, br, N, BN)

    rl = _sc_gather(xl, src, D)
    rr = _sc_gather(xr, dst, D)

    alpha_e = _edge_alpha(rl, rr, eap, WeP, attc)
    alpha_s = _self_alpha(xl, xr, la, WeP, attc)

    alpha = jnp.concatenate([alpha_e, alpha_s], axis=0)
    mx = _amax(alpha)
    ex = _exp(alpha, mx)
    ex_e, ex_s = ex[:E], ex[E:]

    v, exb = _apply(rl, ex_e)
    dn = _sc_scatter_add(exb, dst, 128)
    s0 = _sc_scatter_add(v[:, :D // 2], dst, D // 2)
    s1 = _sc_scatter_add(v[:, D // 2:], dst, D // 2)

    return _final(x, xl, s0, s1, dn, ex_s, cb2d)


def kernel(x, edge_index, edge_attr, batch, smiles, Wl1, bl1, Wr1, br1, We1,
           att1, cb1, Wl2, bl2, Wr2, br2, We2, att2, cb2, L1W, L1b, L2W, L2b):
    src = edge_index[0].astype(jnp.int32)
    dst = edge_index[1].astype(jnp.int32)
    ea = edge_attr.astype(jnp.float32)
    la_vals = jnp.concatenate(
        [ea, jnp.ones((E, 1), jnp.float32), jnp.zeros((E, 116), jnp.float32)], axis=1)
    eap = la_vals
    b2d = batch.astype(jnp.int32).reshape(N, 1)

    def prep(We, att, b_l, b_r, cb):
        return (jnp.pad(We, ((0, 117), (0, 0))), att.reshape(D, 1),
                b_l.reshape(1, D), b_r.reshape(1, D), cb.reshape(1, D))

    WeP1, attc1, bl1r, br1r, cb1r = prep(We1, att1, bl1, br1, cb1)
    WeP2, attc2, bl2r, br2r, cb2r = prep(We2, att2, bl2, br2, cb2)

    la = _sc_scatter_add(la_vals, dst, 128)
    h1 = _gat_layer(x, src, dst, eap, la, Wl1, bl1r, Wr1, br1r, WeP1, attc1, cb1r)
    s1, c1, m1 = _pool(h1, b2d)
    h2 = _gat_layer(h1, src, dst, eap, la, Wl2, bl2r, Wr2, br2r, WeP2, attc2, cb2r)
    s2, c2, m2 = _pool(h2, b2d)

    return _head(s1, c1, m1, s2, c2, m2, L1W, L1b.reshape(1, D), L2W, L2b.reshape(1, 170))


# speedup vs baseline: 3.2586x; 1.2557x over previous
import functools
import jax
import jax.numpy as jnp
from jax import lax
from jax.experimental import pallas as pl
from jax.experimental.pallas import tpu as pltpu
from jax.experimental.pallas import tpu_sc as plsc

N = 10000
E = 160000
G = 64
D = 256

NC = 2
NS = 16
NW = NC * NS

NB = 25
BN = N // NB
EB = 25
BE = E // EB
A = E + N
AB = 25
BA = A // AB

CH = 200
CHS = 200
NPAD = 10240

_mesh = plsc.VectorSubcoreMesh(core_axis_name="c", subcore_axis_name="s")


def _sc_gather(table, idx, d):
    b_per_w = E // NW
    steps = b_per_w // CH

    @functools.partial(
        pl.kernel, mesh=_mesh,
        out_type=jax.ShapeDtypeStruct((E, d), jnp.float32),
        scratch_types=[
            pltpu.VMEM((CH,), jnp.int32),
            pltpu.VMEM((CH, d), jnp.float32),
            pltpu.SemaphoreType.DMA,
        ],
    )
    def k(table_hbm, idx_hbm, out_hbm, idx_v, rows_v, sem):
        wid = lax.axis_index("s") * NC + lax.axis_index("c")

        def body(j, _):
            base = wid * b_per_w + j * CH
            pltpu.sync_copy(idx_hbm.at[pl.ds(base, CH)], idx_v)
            pltpu.async_copy(table_hbm.at[idx_v], rows_v, sem).wait()
            pltpu.sync_copy(rows_v, out_hbm.at[pl.ds(base, CH)])
            return 0

        lax.fori_loop(0, steps, body, 0)

    return k(table, idx)


def _sc_scatter_add(vals, idx, d):
    assert d == 128
    e_per_core = E // NC
    e_per_sub = e_per_core // NS
    steps = e_per_sub // CHS
    rows_per_sub = NPAD // NS

    @functools.partial(
        pl.kernel, mesh=_mesh,
        out_type=jax.ShapeDtypeStruct((NC, NPAD, d), jnp.float32),
        scratch_types=[
            pltpu.VMEM((CHS,), jnp.int32),
            pltpu.VMEM((CHS, d), jnp.float32),
            pltpu.VMEM_SHARED((NPAD, d), jnp.float32),
        ],
    )
    def k(vals_hbm, idx_hbm, zeros_hbm, out_hbm, idx_v, vals_v, acc_sh):
        cid = lax.axis_index("c")
        sid = lax.axis_index("s")

        pltpu.sync_copy(zeros_hbm.at[pl.ds(sid * rows_per_sub, rows_per_sub)],
                        acc_sh.at[pl.ds(sid * rows_per_sub, rows_per_sub)])
        plsc.subcore_barrier()

        def body(j, _):
            base = cid * e_per_core + sid * e_per_sub + j * CHS
            pltpu.sync_copy(idx_hbm.at[pl.ds(base, CHS)], idx_v)
            pltpu.sync_copy(vals_hbm.at[pl.ds(base, CHS)], vals_v)
            pltpu.sync_copy(vals_v, acc_sh.at[idx_v], add=True)
            return 0

        lax.fori_loop(0, steps, body, 0)
        plsc.subcore_barrier()

        pltpu.sync_copy(acc_sh.at[pl.ds(sid * rows_per_sub, rows_per_sub)],
                        out_hbm.at[cid, pl.ds(sid * rows_per_sub, rows_per_sub)])

    zeros = jnp.zeros((NPAD, d), jnp.float32)
    return k(vals, idx, zeros)


def _mm_body(x_ref, w_ref, b_ref, o_ref):
    o_ref[...] = lax.dot_general(
        x_ref[...], w_ref[...], (((1,), (0,)), ((), ())),
        preferred_element_type=jnp.float32) + b_ref[...]


def _mm(x, w, b, rows, block):
    grid = rows // block
    return pl.pallas_call(
        _mm_body,
        grid=(grid,),
        in_specs=[
            pl.BlockSpec((block, w.shape[0]), lambda i: (i, 0)),
            pl.BlockSpec(w.shape, lambda i: (0, 0)),
            pl.BlockSpec((1, w.shape[1]), lambda i: (0, 0)),
        ],
        out_specs=pl.BlockSpec((block, w.shape[1]), lambda i: (i, 0)),
        out_shape=jax.ShapeDtypeStruct((rows, w.shape[1]), jnp.float32),
    )(x, w, b)


def _edge_alpha_body(rl_ref, rr_ref, ea_ref, we_ref, att_ref, o_ref):
    ec = lax.dot_general(ea_ref[...], we_ref[...], (((1,), (0,)), ((), ())),
                         preferred_element_type=jnp.float32)
    m = rl_ref[...] + rr_ref[...] + ec
    m = jnp.where(m > 0, m, 0.2 * m)
    o_ref[...] = lax.dot_general(m, att_ref[...], (((1,), (0,)), ((), ())),
                                 preferred_element_type=jnp.float32)


def _edge_alpha(rl, rr, eap, wep, attc):
    return pl.pallas_call(
        _edge_alpha_body,
        grid=(EB,),
        in_specs=[
            pl.BlockSpec((BE, D), lambda i: (i, 0)),
            pl.BlockSpec((BE, D), lambda i: (i, 0)),
            pl.BlockSpec((BE, 128), lambda i: (i, 0)),
            pl.BlockSpec((128, D), lambda i: (0, 0)),
            pl.BlockSpec((D, 1), lambda i: (0, 0)),
        ],
        out_specs=pl.BlockSpec((BE, 1), lambda i: (i, 0)),
        out_shape=jax.ShapeDtypeStruct((E, 1), jnp.float32),
    )(rl, rr, eap, wep, attc)


def _self_alpha_body(xl_ref, xr_ref, la_ref, we_ref, att_ref, o_ref):
    la = la_ref[0] + la_ref[1]
    deg = jnp.maximum(la[:, 11:12], 1.0)
    lam = la / deg
    colmask = (lax.broadcasted_iota(jnp.int32, (1, 128), 1) < 11).astype(jnp.float32)
    lam = lam * colmask
    ec = lax.dot_general(lam, we_ref[...], (((1,), (0,)), ((), ())),
                         preferred_element_type=jnp.float32)
    m = xl_ref[...] + xr_ref[...] + ec
    m = jnp.where(m > 0, m, 0.2 * m)
    o_ref[...] = lax.dot_general(m, att_ref[...], (((1,), (0,)), ((), ())),
                                 preferred_element_type=jnp.float32)


def _self_alpha(xl, xr, la, wep, attc):
    return pl.pallas_call(
        _self_alpha_body,
        grid=(NB,),
        in_specs=[
            pl.BlockSpec((BN, D), lambda i: (i, 0)),
            pl.BlockSpec((BN, D), lambda i: (i, 0)),
            pl.BlockSpec((NC, BN, 128), lambda i: (0, i, 0)),
            pl.BlockSpec((128, D), lambda i: (0, 0)),
            pl.BlockSpec((D, 1), lambda i: (0, 0)),
        ],
        out_specs=pl.BlockSpec((BN, 1), lambda i: (i, 0)),
        out_shape=jax.ShapeDtypeStruct((N, 1), jnp.float32),
    )(xl, xr, la, wep, attc)


def _amax_body(a_ref, o_ref):
    @pl.when(pl.program_id(0) == 0)
    def _():
        o_ref[...] = jnp.full((1, 1), -1e30, jnp.float32)
    o_ref[...] = jnp.maximum(o_ref[...], jnp.max(a_ref[...])[None, None])


def _amax(a):
    return pl.pallas_call(
        _amax_body,
        grid=(AB,),
        in_specs=[pl.BlockSpec((BA, 1), lambda i: (i, 0))],
        out_specs=pl.BlockSpec((1, 1), lambda i: (0, 0)),
        out_shape=jax.ShapeDtypeStruct((1, 1), jnp.float32),
    )(a)


def _exp_body(a_ref, mx_ref, o_ref):
    o_ref[...] = jnp.exp(a_ref[...] - mx_ref[0, 0])


def _exp(a, mx):
    return pl.pallas_call(
        _exp_body,
        grid=(AB,),
        in_specs=[
            pl.BlockSpec((BA, 1), lambda i: (i, 0)),
            pl.BlockSpec((1, 1), lambda i: (0, 0)),
        ],
        out_specs=pl.BlockSpec((BA, 1), lambda i: (i, 0)),
        out_shape=jax.ShapeDtypeStruct((A, 1), jnp.float32),
    )(a, mx)


def _apply_body(rl_ref, ex_ref, v_ref, exb_ref):
    ex = ex_ref[...]
    v_ref[...] = rl_ref[...] * ex
    col0 = lax.broadcasted_iota(jnp.int32, (BE, 128), 1) == 0
    exb_ref[...] = jnp.where(col0, jnp.broadcast_to(ex, (BE, 128)), 0.0)


def _apply(rl, ex_e):
    return pl.pallas_call(
        _apply_body,
        grid=(EB,),
        in_specs=[
            pl.BlockSpec((BE, D), lambda i: (i, 0)),
            pl.BlockSpec((BE, 1), lambda i: (i, 0)),
        ],
        out_specs=[
            pl.BlockSpec((BE, D), lambda i: (i, 0)),
            pl.BlockSpec((BE, 128), lambda i: (i, 0)),
        ],
        out_shape=[
            jax.ShapeDtypeStruct((E, D), jnp.float32),
            jax.ShapeDtypeStruct((E, 128), jnp.float32),
        ],
    )(rl, ex_e)


def _final_body(x_ref, xl_ref, s0_ref, s1_ref, dn_ref, exs_ref, cb_ref, o_ref):
    seg = jnp.concatenate([s0_ref[0] + s0_ref[1], s1_ref[0] + s1_ref[1]], axis=1)
    exs = exs_ref[...]
    num = seg + exs * xl_ref[...]
    denom = dn_ref[0, :, 0:1] + dn_ref[1, :, 0:1] + exs + 1e-16
    out = num / denom + cb_ref[...]
    h = x_ref[...] + out
    o_ref[...] = jnp.where(h > 0, h, 0.01 * h)


def _final(x, xl, s0, s1, dn, exs, cb2d):
    return pl.pallas_call(
        _final_body,
        grid=(NB,),
        in_specs=[
            pl.BlockSpec((BN, D), lambda i: (i, 0)),
            pl.BlockSpec((BN, D), lambda i: (i, 0)),
            pl.BlockSpec((NC, BN, D // 2), lambda i: (0, i, 0)),
            pl.BlockSpec((NC, BN, D // 2), lambda i: (0, i, 0)),
            pl.BlockSpec((NC, BN, 128), lambda i: (0, i, 0)),
            pl.BlockSpec((BN, 1), lambda i: (i, 0)),
            pl.BlockSpec((1, D), lambda i: (0, 0)),
        ],
        out_specs=pl.BlockSpec((BN, D), lambda i: (i, 0)),
        out_shape=jax.ShapeDtypeStruct((N, D), jnp.float32),
    )(x, xl, s0, s1, dn, exs, cb2d)


def _pool_body(h_ref, b_ref, sum_ref, cnt_ref, max_ref):
    @pl.when(pl.program_id(0) == 0)
    def _():
        sum_ref[...] = jnp.zeros_like(sum_ref)
        cnt_ref[...] = jnp.zeros_like(cnt_ref)
        max_ref[...] = jnp.full_like(max_ref, -1e30)

    b = b_ref[...]
    onehot = (b == lax.broadcasted_iota(jnp.int32, (BN, G), 1)).astype(jnp.float32)
    h = h_ref[...]
    sum_ref[...] += lax.dot_general(onehot, h, (((0,), (0,)), ((), ())),
                                    preferred_element_type=jnp.float32)
    ones = jnp.ones((BN, 1), jnp.float32)
    cnt_ref[...] += lax.dot_general(onehot, ones, (((0,), (0,)), ((), ())),
                                    preferred_element_type=jnp.float32)

    def body(g, _):
        mask = b == g
        mx = jnp.max(jnp.where(mask, h, -1e30), axis=0, keepdims=True)
        max_ref[pl.ds(g, 1), :] = jnp.maximum(max_ref[pl.ds(g, 1), :], mx)
        return 0

    lax.fori_loop(0, G, body, 0)


def _pool(h, b2d):
    return pl.pallas_call(
        _pool_body,
        grid=(NB,),
        in_specs=[
            pl.BlockSpec((BN, D), lambda i: (i, 0)),
            pl.BlockSpec((BN, 1), lambda i: (i, 0)),
        ],
        out_specs=[
            pl.BlockSpec((G, D), lambda i: (0, 0)),
            pl.BlockSpec((G, 1), lambda i: (0, 0)),
            pl.BlockSpec((G, D), lambda i: (0, 0)),
        ],
        out_shape=[
            jax.ShapeDtypeStruct((G, D), jnp.float32),
            jax.ShapeDtypeStruct((G, 1), jnp.float32),
            jax.ShapeDtypeStruct((G, D), jnp.float32),
        ],
    )(h, b2d)


def _head_body(s1_ref, c1_ref, m1_ref, s2_ref, c2_ref, m2_ref,
               w1_ref, b1_ref, w2_ref, b2_ref, o_ref):
    def pooled(s_ref, c_ref, m_ref):
        cnt = c_ref[...]
        mean = s_ref[...] / jnp.maximum(cnt, 1.0)
        mx = jnp.where(cnt > 0, m_ref[...], 0.0)
        return jnp.concatenate([mean, mx], axis=1)

    z = pooled(s1_ref, c1_ref, m1_ref) + pooled(s2_ref, c2_ref, m2_ref)
    z = lax.dot_general(z, w1_ref[...], (((1,), (0,)), ((), ())),
                        preferred_element_type=jnp.float32) + b1_ref[...]
    z = jnp.where(z > 0, z, 0.01 * z)
    o_ref[...] = lax.dot_general(z, w2_ref[...], (((1,), (0,)), ((), ())),
                                 preferred_element_type=jnp.float32) + b2_ref[...]


def _head(s1, c1, m1, s2, c2, m2, w1, b1, w2, b2):
    full = lambda a: pl.BlockSpec(a.shape, lambda: (0,) * a.ndim)
    return pl.pallas_call(
        _head_body,
        in_specs=[full(a) for a in (s1, c1, m1, s2, c2, m2, w1, b1, w2, b2)],
        out_specs=pl.BlockSpec((G, 170), lambda: (0, 0)),
        out_shape=jax.ShapeDtypeStruct((G, 170), jnp.float32),
    )(s1, c1, m1, s2, c2, m2, w1, b1, w2, b2)


def _gat_layer(x, src, dst, eap, la, Wl, bl, Wr, br, WeP, attc, cb2d):
    xl = _mm(x, Wl, bl, N, BN)
    xr = _mm(x, Wr, br, N, BN)

    rl = _sc_gather(xl, src, D)
    rr = _sc_gather(xr, dst, D)

    alpha_e = _edge_alpha(rl, rr, eap, WeP, attc)
    alpha_s = _self_alpha(xl, xr, la, WeP, attc)

    alpha = jnp.concatenate([alpha_e, alpha_s], axis=0)
    mx = _amax(alpha)
    ex = _exp(alpha, mx)
    ex_e, ex_s = ex[:E], ex[E:]

    v, exb = _apply(rl, ex_e)
    dn = _sc_scatter_add(exb, dst, 128)
    s0 = _sc_scatter_add(v[:, :D // 2], dst, D // 2)
    s1 = _sc_scatter_add(v[:, D // 2:], dst, D // 2)

    return _final(x, xl, s0, s1, dn, ex_s, cb2d)


def kernel(x, edge_index, edge_attr, batch, smiles, Wl1, bl1, Wr1, br1, We1,
           att1, cb1, Wl2, bl2, Wr2, br2, We2, att2, cb2, L1W, L1b, L2W, L2b):
    src = edge_index[0].astype(jnp.int32)
    dst = edge_index[1].astype(jnp.int32)
    ea = edge_attr.astype(jnp.float32)
    la_vals = jnp.concatenate(
        [ea, jnp.ones((E, 1), jnp.float32), jnp.zeros((E, 116), jnp.float32)], axis=1)
    eap = la_vals
    b2d = batch.astype(jnp.int32).reshape(N, 1)

    def prep(We, att, b_l, b_r, cb):
        return (jnp.pad(We, ((0, 117), (0, 0))), att.reshape(D, 1),
                b_l.reshape(1, D), b_r.reshape(1, D), cb.reshape(1, D))

    WeP1, attc1, bl1r, br1r, cb1r = prep(We1, att1, bl1, br1, cb1)
    WeP2, attc2, bl2r, br2r, cb2r = prep(We2, att2, bl2, br2, cb2)

    la = _sc_scatter_add(la_vals, dst, 128)
    h1 = _gat_layer(x, src, dst, eap, la, Wl1, bl1r, Wr1, br1r, WeP1, attc1, cb1r)
    s1, c1, m1 = _pool(h1, b2d)
    h2 = _gat_layer(h1, src, dst, eap, la, Wl2, bl2r, Wr2, br2r, WeP2, attc2, cb2r)
    s2, c2, m2 = _pool(h2, b2d)

    return _head(s1, c1, m1, s2, c2, m2, L1W, L1b.reshape(1, D), L2W, L2b.reshape(1, 170))
